# baseline (speedup 1.0000x reference)
import functools

import jax
from jax import lax
import jax.numpy as jnp
from jax.experimental import pallas as pl
from jax.experimental.pallas import tpu as pltpu
from jax.experimental.pallas import tpu_sc as plsc

B = 1024
NIN = 2048
NHIDDEN = 8192
NOUT = 2048
K = 64

HBE = 1024
BRM = 256
KBD = 1024


def _enc_body(x_ref, w_ref, b_ref, o_ref):
    acc = jax.lax.dot_general(
        x_ref[...], w_ref[...], (((1,), (1,)), ((), ())),
        preferred_element_type=jnp.float32)
    o_ref[...] = acc + b_ref[...]


def _key(h):
    iv = jax.lax.bitcast_convert_type(h, jnp.uint32)
    return jnp.where((iv >> 31) != 0, ~iv, iv | jnp.uint32(0x80000000))


NC = 2
NS = 16
NW = NC * NS
RPW = B // NW
NV = NHIDDEN // 16


def _splat(v):
    return jnp.full((16,), v, jnp.int32)


def _sc_key16(v):
    iv = lax.bitcast_convert_type(v, jnp.uint32)
    return jnp.where((iv >> 31) != 0, ~iv, iv | jnp.uint32(0x80000000))


def _sc_scan(hist_ref, thresh, lane, zeros):
    found = jnp.zeros((16,), jnp.bool_)
    bucket = zeros
    cnt_ge = zeros
    hist_b = zeros
    carry = zeros
    for g in range(15, -1, -1):
        v = hist_ref[pl.ds(g * 16, 16)]
        rv = lax.rev(v, (0,))
        cs = plsc.cumsum(rv) + carry
        hit = cs >= thresh
        npop = plsc.all_reduce_population_count(hit)
        ffs = plsc.all_reduce_ffs(hit)
        cg = _splat(jnp.min(jnp.where(hit, cs, _splat(1 << 30))))
        hb = _splat(jnp.max(jnp.where(lane == ffs, rv, zeros)))
        this_found = npop > 0
        upd = jnp.logical_and(this_found, jnp.logical_not(found))
        bucket = jnp.where(upd, _splat(g * 16 + 15) - ffs, bucket)
        cnt_ge = jnp.where(upd, cg, cnt_ge)
        hist_b = jnp.where(upd, hb, hist_b)
        found = jnp.logical_or(found, this_found)
        carry = carry + _splat(jnp.sum(v))
    return bucket, thresh - (cnt_ge - hist_b)


def _sc_thr_body(h_hbm, thr_hbm, rowa_v, rowb_v, hp_v, hist_v, cand_v,
                 thr_v, sema, semb):
    wid = lax.axis_index("s") * NC + lax.axis_index("c")
    base = wid * RPW
    lane = lax.iota(jnp.int32, 16)
    ones = _splat(1)
    zeros = _splat(0)
    lane257 = lane * 257
    lane513 = lane * 513

    def fetch(ri, dst, sem):
        ri = jnp.minimum(ri, RPW - 1)
        pltpu.make_async_copy(
            h_hbm.at[pl.ds((base + ri) * NHIDDEN, NHIDDEN)], dst, sem).start()

    def process(row_v, r):
        def zero_hp(i, c):
            for j in range(8):
                hp_v[pl.ds((i * 8 + j) * 16, 16)] = zeros
            return c

        lax.fori_loop(0, 257 * 16 // (16 * 8), zero_hp, 0)
        hp_v[pl.ds(256 * 16, 16)] = zeros

        def p1(i, c):
            for j in range(8):
                u = _sc_key16(row_v[pl.ds((i * 8 + j) * 16, 16)])
                b = (u >> 24).astype(jnp.int32)
                plsc.addupdate_scatter(hp_v, [lane257 + b], ones)
            return c

        lax.fori_loop(0, NV // 8, p1, 0)

        for g in range(16):
            acc = hp_v[pl.ds(g * 16, 16)]
            for l in range(1, 16):
                acc = acc + hp_v[pl.ds(l * 257 + g * 16, 16)]
            hist_v[pl.ds(g * 16, 16)] = acc
        bucket, need = _sc_scan(hist_v, _splat(K), lane, zeros)
        bucket_hi = lax.shift_left(bucket.astype(jnp.uint32), jnp.uint32(24))

        def p2(i, cnt):
            for j in range(8):
                u = _sc_key16(row_v[pl.ds((i * 8 + j) * 16, 16)])
                m = (u >> 24).astype(jnp.int32) == bucket
                plsc.store_scatter(
                    cand_v, [lane513 + cnt],
                    lax.bitcast_convert_type(u, jnp.int32), mask=m)
                cnt = cnt + m.astype(jnp.int32)
            return cnt

        cnt = lax.fori_loop(0, NV // 8, p2, zeros)
        maxc = jnp.max(cnt)

        def bstep(t, tl):
            cand_t = tl | lax.shift_left(
                jnp.uint32(1), jnp.uint32(23) - t.astype(jnp.uint32))
            target = bucket_hi | cand_t

            def cl(j, acc):
                g = plsc.load_gather(cand_v, [lane513 + _splat(j)])
                uu = lax.bitcast_convert_type(g, jnp.uint32)
                m = jnp.logical_and(_splat(j) < cnt, uu >= target)
                return acc + m.astype(jnp.int32)

            pc = lax.fori_loop(0, maxc, cl, zeros)
            total = _splat(jnp.sum(pc))
            return jnp.where(total >= need, cand_t, tl)

        tl = lax.fori_loop(0, 24, bstep, jnp.zeros((16,), jnp.uint32))
        key = lax.bitcast_convert_type(bucket_hi | tl, jnp.int32)
        plsc.store_scatter(thr_v, [_splat(r)], key, mask=lane == 0)

    fetch(0, rowa_v, sema)
    def pair(p, c):
        fetch(2 * p + 1, rowb_v, semb)
        pltpu.make_async_copy(h_hbm.at[pl.ds(0, NHIDDEN)], rowa_v, sema).wait()
        process(rowa_v, 2 * p)
        fetch(2 * p + 2, rowa_v, sema)
        pltpu.make_async_copy(h_hbm.at[pl.ds(0, NHIDDEN)], rowb_v, semb).wait()
        process(rowb_v, 2 * p + 1)
        return c

    lax.fori_loop(0, RPW // 2, pair, 0)
    pltpu.make_async_copy(h_hbm.at[pl.ds(0, NHIDDEN)], rowa_v, sema).wait()
    pltpu.sync_copy(thr_v, thr_hbm.at[pl.ds(base, RPW)])


def _dec_body(h_ref, t_ref, w_ref, b_ref, o_ref):
    k = pl.program_id(0)

    @pl.when(k == 0)
    def _():
        o_ref[...] = jnp.broadcast_to(b_ref[...], o_ref.shape)

    h = h_ref[...]
    thr = jax.lax.bitcast_convert_type(t_ref[...], jnp.uint32)
    hm = jnp.where(_key(h) >= thr, h, 0.0)
    o_ref[...] += jax.lax.dot_general(
        hm, w_ref[...], (((1,), (1,)), ((), ())),
        preferred_element_type=jnp.float32)


def kernel(x, W_enc, b_enc, W_dec, b_dec):
    h = pl.pallas_call(
        _enc_body,
        grid=(NHIDDEN // HBE,),
        in_specs=[
            pl.BlockSpec((B, NIN), lambda j: (0, 0)),
            pl.BlockSpec((HBE, NIN), lambda j: (j, 0)),
            pl.BlockSpec((1, HBE), lambda j: (0, j)),
        ],
        out_specs=pl.BlockSpec((B, HBE), lambda j: (0, j)),
        out_shape=jax.ShapeDtypeStruct((B, NHIDDEN), jnp.float32),
    )(x, W_enc, b_enc.reshape(1, NHIDDEN))

    thr = pl.kernel(
        _sc_thr_body,
        out_type=jax.ShapeDtypeStruct((B,), jnp.int32),
        mesh=plsc.VectorSubcoreMesh(core_axis_name="c", subcore_axis_name="s"),
        scratch_types=[
            pltpu.VMEM((NHIDDEN,), jnp.float32),
            pltpu.VMEM((NHIDDEN,), jnp.float32),
            pltpu.VMEM((257 * 16,), jnp.int32),
            pltpu.VMEM((256,), jnp.int32),
            pltpu.VMEM((513 * 16,), jnp.int32),
            pltpu.VMEM((RPW,), jnp.int32),
            pltpu.SemaphoreType.DMA,
            pltpu.SemaphoreType.DMA,
        ],
        compiler_params=pltpu.CompilerParams(needs_layout_passes=False),
    )(h.reshape(-1))

    out = pl.pallas_call(
        _dec_body,
        grid=(NHIDDEN // KBD,),
        in_specs=[
            pl.BlockSpec((B, KBD), lambda k: (0, k)),
            pl.BlockSpec((B, 1), lambda k: (0, 0)),
            pl.BlockSpec((NOUT, KBD), lambda k: (0, k)),
            pl.BlockSpec((1, NOUT), lambda k: (0, 0)),
        ],
        out_specs=pl.BlockSpec((B, NOUT), lambda k: (0, 0)),
        out_shape=jax.ShapeDtypeStruct((B, NOUT), jnp.float32),
    )(h, thr.reshape(B, 1), W_dec, b_dec.reshape(1, NOUT))
    return out


# speedup vs baseline: 2.8692x; 2.8692x over previous
import jax
import jax.numpy as jnp
from jax.experimental import pallas as pl
from jax.experimental.pallas import tpu as pltpu

B = 1024
NIN = 2048
NHIDDEN = 8192
NOUT = 2048
K = 64

HBE = 1024
BRM = 256
KBD = 1024


def _enc_body(x_ref, w_ref, b_ref, o_ref):
    acc = jax.lax.dot_general(
        x_ref[...], w_ref[...], (((1,), (1,)), ((), ())),
        preferred_element_type=jnp.float32)
    o_ref[...] = acc + b_ref[...]


def _key(h):
    iv = jax.lax.bitcast_convert_type(h, jnp.uint32)
    return jnp.where((iv >> 31) != 0, ~iv, iv | jnp.uint32(0x80000000))


def _thr_body(h_ref, t_ref):
    h = h_ref[...]
    br = h.shape[0]
    u = _key(h)

    def step(t, thr):
        s = jnp.uint32(30) - 2 * t.astype(jnp.uint32)
        c1 = thr | jax.lax.shift_left(jnp.uint32(1), s)
        c2 = thr | jax.lax.shift_left(jnp.uint32(2), s)
        c3 = thr | jax.lax.shift_left(jnp.uint32(3), s)
        n1 = jnp.sum((u >= c1).astype(jnp.int32), axis=1, keepdims=True)
        n2 = jnp.sum((u >= c2).astype(jnp.int32), axis=1, keepdims=True)
        n3 = jnp.sum((u >= c3).astype(jnp.int32), axis=1, keepdims=True)
        thr = jnp.where(n1 >= K, c1, thr)
        thr = jnp.where(n2 >= K, c2, thr)
        thr = jnp.where(n3 >= K, c3, thr)
        return thr

    thr = jax.lax.fori_loop(0, 16, step, jnp.zeros((br, 1), jnp.uint32))
    t_ref[...] = jnp.broadcast_to(thr, (br, 128))


def _dec_body(h_ref, t_ref, w_ref, b_ref, o_ref):
    k = pl.program_id(0)

    @pl.when(k == 0)
    def _():
        o_ref[...] = jnp.broadcast_to(b_ref[...], o_ref.shape)

    h = h_ref[...]
    hm = jnp.where(_key(h) >= t_ref[:, 0:1], h, 0.0)
    o_ref[...] += jax.lax.dot_general(
        hm, w_ref[...], (((1,), (1,)), ((), ())),
        preferred_element_type=jnp.float32)


def kernel(x, W_enc, b_enc, W_dec, b_dec):
    h = pl.pallas_call(
        _enc_body,
        grid=(NHIDDEN // HBE,),
        in_specs=[
            pl.BlockSpec((B, NIN), lambda j: (0, 0)),
            pl.BlockSpec((HBE, NIN), lambda j: (j, 0)),
            pl.BlockSpec((1, HBE), lambda j: (0, j)),
        ],
        out_specs=pl.BlockSpec((B, HBE), lambda j: (0, j)),
        out_shape=jax.ShapeDtypeStruct((B, NHIDDEN), jnp.float32),
    )(x, W_enc, b_enc.reshape(1, NHIDDEN))

    thr = pl.pallas_call(
        _thr_body,
        grid=(B // BRM,),
        in_specs=[pl.BlockSpec((BRM, NHIDDEN), lambda i: (i, 0))],
        out_specs=pl.BlockSpec((BRM, 128), lambda i: (i, 0)),
        out_shape=jax.ShapeDtypeStruct((B, 128), jnp.uint32),
    )(h)

    out = pl.pallas_call(
        _dec_body,
        grid=(NHIDDEN // KBD,),
        in_specs=[
            pl.BlockSpec((B, KBD), lambda k: (0, k)),
            pl.BlockSpec((B, 128), lambda k: (0, 0)),
            pl.BlockSpec((NOUT, KBD), lambda k: (0, k)),
            pl.BlockSpec((1, NOUT), lambda k: (0, 0)),
        ],
        out_specs=pl.BlockSpec((B, NOUT), lambda k: (0, 0)),
        out_shape=jax.ShapeDtypeStruct((B, NOUT), jnp.float32),
    )(h, thr, W_dec, b_dec.reshape(1, NOUT))
    return out


# speedup vs baseline: 3.4434x; 1.2001x over previous
import jax
import jax.numpy as jnp
from jax.experimental import pallas as pl
from jax.experimental.pallas import tpu as pltpu

B = 1024
NIN = 2048
NHIDDEN = 8192
NOUT = 2048
K = 64

HBE = 1024
BRM = 256
KBD = 1024


def _enc_body(x_ref, w_ref, b_ref, o_ref):
    acc = jax.lax.dot_general(
        x_ref[...], w_ref[...], (((1,), (1,)), ((), ())),
        preferred_element_type=jnp.float32)
    o_ref[...] = acc + b_ref[...]


def _key(h):
    iv = jax.lax.bitcast_convert_type(h, jnp.uint32)
    return jnp.where((iv >> 31) != 0, ~iv, iv | jnp.uint32(0x80000000))


def _thr_body(h_ref, t_ref):
    h = h_ref[...]
    br = h.shape[0]
    u = _key(h)

    def step(t, thr):
        cand = thr | jax.lax.shift_left(
            jnp.uint32(1), jnp.uint32(31) - t.astype(jnp.uint32))
        cnt = jnp.sum((u >= cand).astype(jnp.int32), axis=1, keepdims=True)
        return jnp.where(cnt >= K, cand, thr)

    thr = jax.lax.fori_loop(0, 32, step, jnp.zeros((br, 1), jnp.uint32))
    t_ref[...] = jnp.broadcast_to(thr, (br, 128))


def _dec_body(h_ref, t_ref, w_ref, b_ref, o_ref):
    k = pl.program_id(0)

    @pl.when(k == 0)
    def _():
        o_ref[...] = jnp.broadcast_to(b_ref[...], o_ref.shape)

    h = h_ref[...]
    hm = jnp.where(_key(h) >= t_ref[:, 0:1], h, 0.0)
    o_ref[...] += jax.lax.dot_general(
        hm, w_ref[...], (((1,), (1,)), ((), ())),
        preferred_element_type=jnp.float32)


def kernel(x, W_enc, b_enc, W_dec, b_dec):
    h = pl.pallas_call(
        _enc_body,
        grid=(NHIDDEN // HBE,),
        in_specs=[
            pl.BlockSpec((B, NIN), lambda j: (0, 0)),
            pl.BlockSpec((HBE, NIN), lambda j: (j, 0)),
            pl.BlockSpec((1, HBE), lambda j: (0, j)),
        ],
        out_specs=pl.BlockSpec((B, HBE), lambda j: (0, j)),
        out_shape=jax.ShapeDtypeStruct((B, NHIDDEN), jnp.float32),
    )(x, W_enc, b_enc.reshape(1, NHIDDEN))

    thr = pl.pallas_call(
        _thr_body,
        grid=(B // BRM,),
        in_specs=[pl.BlockSpec((BRM, NHIDDEN), lambda i: (i, 0))],
        out_specs=pl.BlockSpec((BRM, 128), lambda i: (i, 0)),
        out_shape=jax.ShapeDtypeStruct((B, 128), jnp.uint32),
    )(h)

    out = pl.pallas_call(
        _dec_body,
        grid=(NHIDDEN // KBD,),
        in_specs=[
            pl.BlockSpec((B, KBD), lambda k: (0, k)),
            pl.BlockSpec((B, 128), lambda k: (0, 0)),
            pl.BlockSpec((NOUT, KBD), lambda k: (0, k)),
            pl.BlockSpec((1, NOUT), lambda k: (0, 0)),
        ],
        out_specs=pl.BlockSpec((B, NOUT), lambda k: (0, 0)),
        out_shape=jax.ShapeDtypeStruct((B, NOUT), jnp.float32),
    )(h, thr, W_dec, b_dec.reshape(1, NOUT))
    return out
